# baseline (speedup 1.0000x reference)
import jax
import jax.numpy as jnp
from jax.experimental import pallas as pl

_L_BLOCK = 1024
_B_BLOCK = 1


def _add_body(x_ref, t_ref, o_ref):
    o_ref[...] = x_ref[...] + t_ref[...][None, :, :]


def kernel(x, pos_table):
    B, L, D = x.shape
    lb = min(_L_BLOCK, L)
    bb = min(_B_BLOCK, B)
    return pl.pallas_call(
        _add_body,
        grid=(L // lb, B // bb),
        in_specs=[
            pl.BlockSpec((bb, lb, D), lambda i, j: (j, i, 0)),
            pl.BlockSpec((lb, D), lambda i, j: (i, 0)),
        ],
        out_specs=pl.BlockSpec((bb, lb, D), lambda i, j: (j, i, 0)),
        out_shape=jax.ShapeDtypeStruct((B, L, D), x.dtype),
    )(x, pos_table[:L])


# speedup vs baseline: 1.0205x; 1.0205x over previous
import jax
import jax.numpy as jnp
from jax.experimental import pallas as pl

_L_BLOCK = 512
_B_BLOCK = 2


def _add_body(x_ref, t_ref, o_ref):
    o_ref[...] = x_ref[...] + t_ref[...][None, :, :]


def kernel(x, pos_table):
    B, L, D = x.shape
    lb = min(_L_BLOCK, L)
    bb = min(_B_BLOCK, B)
    return pl.pallas_call(
        _add_body,
        grid=(L // lb, B // bb),
        in_specs=[
            pl.BlockSpec((bb, lb, D), lambda i, j: (j, i, 0)),
            pl.BlockSpec((lb, D), lambda i, j: (i, 0)),
        ],
        out_specs=pl.BlockSpec((bb, lb, D), lambda i, j: (j, i, 0)),
        out_shape=jax.ShapeDtypeStruct((B, L, D), x.dtype),
    )(x, pos_table[:L])


# speedup vs baseline: 1.0698x; 1.0483x over previous
import jax
import jax.numpy as jnp
from jax.experimental import pallas as pl

_L_BLOCK = 1024
_B_BLOCK = 2


def _add_body(x_ref, t_ref, o_ref):
    o_ref[...] = x_ref[...] + t_ref[...][None, :, :]


def kernel(x, pos_table):
    B, L, D = x.shape
    lb = min(_L_BLOCK, L)
    bb = min(_B_BLOCK, B)
    return pl.pallas_call(
        _add_body,
        grid=(L // lb, B // bb),
        in_specs=[
            pl.BlockSpec((bb, lb, D), lambda i, j: (j, i, 0)),
            pl.BlockSpec((lb, D), lambda i, j: (i, 0)),
        ],
        out_specs=pl.BlockSpec((bb, lb, D), lambda i, j: (j, i, 0)),
        out_shape=jax.ShapeDtypeStruct((B, L, D), x.dtype),
    )(x, pos_table[:L])
